# Initial kernel scaffold; baseline (speedup 1.0000x reference)
import jax
import jax.numpy as jnp
from jax.experimental import pallas as pl


def kernel(x, edge_index, W0, att_src0, att_dst0, bias0, ln_g, ln_b, W1, att_src1, att_dst1, bias1):
    raise NotImplementedError("write your pallas kernel here")


# speedup vs baseline: 33.9985x; 33.9985x over previous
import functools

import jax
import jax.numpy as jnp
from jax import lax
from jax.experimental import pallas as pl
from jax.experimental.pallas import tpu as pltpu
from jax.experimental.pallas import tpu_sc as plsc

_N = 10000
_D = 128
_E = 320000
_RB = 2000
_NW = 32
_EPW = _E // _NW
_B = 80
_NB = _EPW // _B
_NACC = 10240
_RPT = _NACC // 16
_RZ = 128


def _prep_body(x_ref, w_ref, s_ref, d_ref, xl_out, as_out, ad_out):
    xl = jnp.dot(x_ref[:], w_ref[:], preferred_element_type=jnp.float32)
    xl_out[:] = xl
    as_out[:] = jnp.dot(xl, s_ref[:], preferred_element_type=jnp.float32)
    ad_out[:] = jnp.dot(xl, d_ref[:], preferred_element_type=jnp.float32)


def _prep(x, W, S, Dm):
    grid = (_N // _RB,)
    return pl.pallas_call(
        _prep_body,
        grid=grid,
        in_specs=[
            pl.BlockSpec((_RB, _D), lambda i: (i, 0)),
            pl.BlockSpec((_D, _D), lambda i: (0, 0)),
            pl.BlockSpec((_D, 16), lambda i: (0, 0)),
            pl.BlockSpec((_D, 16), lambda i: (0, 0)),
        ],
        out_specs=[
            pl.BlockSpec((_RB, _D), lambda i: (i, 0)),
            pl.BlockSpec((_RB, 16), lambda i: (i, 0)),
            pl.BlockSpec((_RB, 16), lambda i: (i, 0)),
        ],
        out_shape=[
            jax.ShapeDtypeStruct((_N, _D), jnp.float32),
            jax.ShapeDtypeStruct((_N, 16), jnp.float32),
            jax.ShapeDtypeStruct((_N, 16), jnp.float32),
        ],
    )(x, W, S, Dm)


@functools.lru_cache(maxsize=None)
def _make_edge_pass(maps):
    mesh = plsc.VectorSubcoreMesh(core_axis_name="c", subcore_axis_name="s")

    @functools.partial(
        pl.kernel,
        mesh=mesh,
        compiler_params=pltpu.CompilerParams(use_tc_tiling_on_sc=False),
        out_type=[
            jax.ShapeDtypeStruct((2, _NACC, _D), jnp.float32),
            jax.ShapeDtypeStruct((2, _NACC, 16), jnp.float32),
        ],
        scratch_types=[
            pltpu.VMEM((_B,), jnp.int32),
            pltpu.VMEM((_B,), jnp.int32),
            pltpu.VMEM((_B, 16), jnp.float32),
            pltpu.VMEM((_B, 16), jnp.float32),
            pltpu.VMEM((_B, _D), jnp.float32),
            pltpu.VMEM((_B, 16), jnp.float32),
            pltpu.VMEM((_RZ, _D), jnp.float32),
            pltpu.VMEM((_RZ, 16), jnp.float32),
            pltpu.VMEM_SHARED((_NACC, _D), jnp.float32),
            pltpu.VMEM_SHARED((_NACC, 16), jnp.float32),
            pltpu.SemaphoreType.DMA,
        ],
    )
    def k(xl_hbm, asrc_hbm, adst_hbm, sidx_hbm, didx_hbm,
          accm_hbm, accw_hbm,
          sidx_v, didx_v, gs_v, gd_v, msg_v, w_v, zm_v, zw_v,
          accm_sh, accw_sh, sem):
        cid = lax.axis_index("c")
        sid = lax.axis_index("s")
        wid = cid * 16 + sid

        zero16 = jnp.zeros((16,), jnp.float32)

        def zrow(r, c):
            for kk in range(8):
                zm_v[r, pl.ds(kk * 16, 16)] = zero16
            zw_v[r, :] = zero16
            return c

        lax.fori_loop(0, _RZ, zrow, 0)
        rbase = sid * _RPT
        for j in range(_RPT // _RZ):
            pltpu.sync_copy(zm_v, accm_sh.at[pl.ds(rbase + j * _RZ, _RZ)])
            pltpu.sync_copy(zw_v, accw_sh.at[pl.ds(rbase + j * _RZ, _RZ)])
        plsc.subcore_barrier()

        ebase = wid * _EPW

        def block(i, c):
            off = ebase + i * _B
            pltpu.sync_copy(sidx_hbm.at[pl.ds(off, _B)], sidx_v)
            pltpu.sync_copy(didx_hbm.at[pl.ds(off, _B)], didx_v)
            pltpu.async_copy(xl_hbm.at[sidx_v], msg_v, sem).wait()
            pltpu.async_copy(asrc_hbm.at[sidx_v], gs_v, sem).wait()
            pltpu.async_copy(adst_hbm.at[didx_v], gd_v, sem).wait()

            def edge(e, c2):
                a = gs_v[e, :] + gd_v[e, :]
                a = jnp.where(a >= 0.0, a, a * 0.2)
                w = jnp.exp(a)
                w_v[e, :] = w
                ws = None
                for kk in range(8):
                    if kk == 0 or maps[kk] != maps[kk - 1]:
                        ws = jnp.broadcast_to(w[maps[kk]], (16,))
                    sl = pl.ds(kk * 16, 16)
                    msg_v[e, sl] = msg_v[e, sl] * ws
                return c2

            lax.fori_loop(0, _B, edge, 0)
            pltpu.sync_copy(msg_v, accm_sh.at[didx_v], add=True)
            pltpu.sync_copy(w_v, accw_sh.at[didx_v], add=True)
            return c

        lax.fori_loop(0, _NB, block, 0)
        plsc.subcore_barrier()

        for j in range(_RPT // _RZ):
            r0 = rbase + j * _RZ
            pltpu.sync_copy(accm_sh.at[pl.ds(r0, _RZ)],
                            accm_hbm.at[cid, pl.ds(r0, _RZ)])
            pltpu.sync_copy(accw_sh.at[pl.ds(r0, _RZ)],
                            accw_hbm.at[cid, pl.ds(r0, _RZ)])

    return k


def _mid_body(x_ref, xl_ref, as_ref, ad_ref, aa_ref, ab_ref, da_ref, db_ref,
              b0_ref, g_ref, be_ref, w1_ref, s1_ref, d1_ref, p_ref,
              xl1_out, as1_out, ad1_out):
    a16 = as_ref[:] + ad_ref[:]
    wself = jnp.exp(jnp.where(a16 >= 0.0, a16, a16 * 0.2))
    den16 = da_ref[:] + db_ref[:] + wself
    wexp = jnp.dot(wself, p_ref[:], preferred_element_type=jnp.float32)
    dexp = jnp.dot(den16, p_ref[:], preferred_element_type=jnp.float32)
    num = aa_ref[:] + ab_ref[:] + wexp * xl_ref[:]
    g = num / dexp + b0_ref[:]
    mu = jnp.mean(g, axis=1, keepdims=True)
    var = jnp.mean((g - mu) ** 2, axis=1, keepdims=True)
    ln = (g - mu) / jnp.sqrt(var + 1e-5) * g_ref[:] + be_ref[:]
    el = jnp.where(ln > 0.0, ln, jnp.exp(ln) - 1.0)
    h = el + x_ref[:]
    xl1 = jnp.dot(h, w1_ref[:], preferred_element_type=jnp.float32)
    xl1_out[:] = xl1
    as1_out[:] = jnp.dot(xl1, s1_ref[:], preferred_element_type=jnp.float32)
    ad1_out[:] = jnp.dot(xl1, d1_ref[:], preferred_element_type=jnp.float32)


def _mid(x, xl0, as0, ad0, aa, ab, da, db, b0, g, be, W1, S1, D1, P):
    grid = (_N // _RB,)
    rb = lambda i: (i, 0)
    z = lambda i: (0, 0)
    return pl.pallas_call(
        _mid_body,
        grid=grid,
        in_specs=[
            pl.BlockSpec((_RB, _D), rb), pl.BlockSpec((_RB, _D), rb),
            pl.BlockSpec((_RB, 16), rb), pl.BlockSpec((_RB, 16), rb),
            pl.BlockSpec((_RB, _D), rb), pl.BlockSpec((_RB, _D), rb),
            pl.BlockSpec((_RB, 16), rb), pl.BlockSpec((_RB, 16), rb),
            pl.BlockSpec((1, _D), z), pl.BlockSpec((1, _D), z),
            pl.BlockSpec((1, _D), z),
            pl.BlockSpec((_D, _D), z),
            pl.BlockSpec((_D, 16), z), pl.BlockSpec((_D, 16), z),
            pl.BlockSpec((16, _D), z),
        ],
        out_specs=[
            pl.BlockSpec((_RB, _D), rb),
            pl.BlockSpec((_RB, 16), rb),
            pl.BlockSpec((_RB, 16), rb),
        ],
        out_shape=[
            jax.ShapeDtypeStruct((_N, _D), jnp.float32),
            jax.ShapeDtypeStruct((_N, 16), jnp.float32),
            jax.ShapeDtypeStruct((_N, 16), jnp.float32),
        ],
    )(x, xl0, as0, ad0, aa, ab, da, db, b0, g, be, W1, S1, D1, P)


def _fin_body(xl1_ref, as_ref, ad_ref, aa_ref, ab_ref, da_ref, db_ref,
              b1_ref, pf_ref, out_ref):
    a16 = as_ref[:] + ad_ref[:]
    wself = jnp.exp(jnp.where(a16 >= 0.0, a16, a16 * 0.2))
    den16 = da_ref[:] + db_ref[:] + wself
    w128 = jnp.dot(wself, pf_ref[:], preferred_element_type=jnp.float32)
    d128 = jnp.dot(den16, pf_ref[:], preferred_element_type=jnp.float32)
    num = aa_ref[:] + ab_ref[:] + w128 * xl1_ref[:]
    out_ref[:] = num / d128 + b1_ref[:]


def _fin(xl1, as1, ad1, aa, ab, da, db, b1, PF):
    grid = (_N // _RB,)
    rb = lambda i: (i, 0)
    z = lambda i: (0, 0)
    return pl.pallas_call(
        _fin_body,
        grid=grid,
        in_specs=[
            pl.BlockSpec((_RB, _D), rb),
            pl.BlockSpec((_RB, 16), rb), pl.BlockSpec((_RB, 16), rb),
            pl.BlockSpec((_RB, _D), rb), pl.BlockSpec((_RB, _D), rb),
            pl.BlockSpec((_RB, 16), rb), pl.BlockSpec((_RB, 16), rb),
            pl.BlockSpec((1, _D), z), pl.BlockSpec((16, _D), z),
        ],
        out_specs=pl.BlockSpec((_RB, _D), rb),
        out_shape=jax.ShapeDtypeStruct((_N, _D), jnp.float32),
    )(xl1, as1, ad1, aa, ab, da, db, b1, PF)


def kernel(x, edge_index, W0, att_src0, att_dst0, bias0, ln_g, ln_b,
           W1, att_src1, att_dst1, bias1):
    ei = edge_index.astype(jnp.int32)
    sidx, didx = ei[0], ei[1]

    eye8 = jnp.eye(8, 16, dtype=jnp.float32)
    S0 = (att_src0.reshape(8, 16)[:, :, None] * eye8[:, None, :]).reshape(128, 16)
    D0 = (att_dst0.reshape(8, 16)[:, :, None] * eye8[:, None, :]).reshape(128, 16)
    S1 = jnp.pad(att_src1.reshape(128, 1), ((0, 0), (0, 15)))
    D1 = jnp.pad(att_dst1.reshape(128, 1), ((0, 0), (0, 15)))
    P = jnp.repeat(jnp.eye(16, dtype=jnp.float32)[:, :8], 16, axis=1)
    PF = jnp.zeros((16, _D), jnp.float32).at[0].set(1.0)

    b0 = bias0.reshape(1, _D)
    b1 = bias1.reshape(1, _D)
    g = ln_g.reshape(1, _D)
    be = ln_b.reshape(1, _D)

    xl0, as0, ad0 = _prep(x, W0, S0, D0)
    accm0, accw0 = _make_edge_pass((0, 1, 2, 3, 4, 5, 6, 7))(
        xl0, as0, ad0, sidx, didx)
    xl1, as1, ad1 = _mid(x, xl0, as0, ad0,
                         accm0[0, :_N], accm0[1, :_N],
                         accw0[0, :_N], accw0[1, :_N],
                         b0, g, be, W1, S1, D1, P)
    accm1, accw1 = _make_edge_pass((0, 0, 0, 0, 0, 0, 0, 0))(
        xl1, as1, ad1, sidx, didx)
    return _fin(xl1, as1, ad1, accm1[0, :_N], accm1[1, :_N],
                accw1[0, :_N], accw1[1, :_N], b1, PF)


# speedup vs baseline: 48.2626x; 1.4196x over previous
import functools

import jax
import jax.numpy as jnp
from jax import lax
from jax.experimental import pallas as pl
from jax.experimental.pallas import tpu as pltpu
from jax.experimental.pallas import tpu_sc as plsc

_N = 10000
_D = 128
_E = 320000
_RB = 2000
_NW = 32
_EPW = _E // _NW
_B = 100
_NB = _EPW // _B
_DF = _D + 16
_NACC = 10240
_RPT = _NACC // 16
_RZ = 160


def _prep_body(x_ref, w_ref, s_ref, d_ref, t_out, ad_out):
    xl = jnp.dot(x_ref[:], w_ref[:], preferred_element_type=jnp.float32)
    t_out[:, :_D] = xl
    t_out[:, _D:] = jnp.dot(xl, s_ref[:], preferred_element_type=jnp.float32)
    ad_out[:] = jnp.dot(xl, d_ref[:], preferred_element_type=jnp.float32)


def _prep(x, W, S, Dm):
    grid = (_N // _RB,)
    rb = lambda i: (i, 0)
    z = lambda i: (0, 0)
    return pl.pallas_call(
        _prep_body,
        grid=grid,
        in_specs=[
            pl.BlockSpec((_RB, _D), rb),
            pl.BlockSpec((_D, _D), z),
            pl.BlockSpec((_D, 16), z),
            pl.BlockSpec((_D, 16), z),
        ],
        out_specs=[
            pl.BlockSpec((_RB, _DF), rb),
            pl.BlockSpec((_RB, 16), rb),
        ],
        out_shape=[
            jax.ShapeDtypeStruct((_N, _DF), jnp.float32),
            jax.ShapeDtypeStruct((_N, 16), jnp.float32),
        ],
    )(x, W, S, Dm)


@functools.lru_cache(maxsize=None)
def _make_edge_pass(maps):
    mesh = plsc.VectorSubcoreMesh(core_axis_name="c", subcore_axis_name="s")

    @functools.partial(
        pl.kernel,
        mesh=mesh,
        compiler_params=pltpu.CompilerParams(use_tc_tiling_on_sc=False),
        out_type=jax.ShapeDtypeStruct((2, _NACC, _DF), jnp.float32),
        scratch_types=[
            pltpu.VMEM((_NB, _B), jnp.int32),
            pltpu.VMEM((_NB, _B), jnp.int32),
            pltpu.VMEM((_B, _DF), jnp.float32),
            pltpu.VMEM((_B, 16), jnp.float32),
            pltpu.VMEM_SHARED((_NACC, _DF), jnp.float32),
            pltpu.SemaphoreType.DMA,
        ],
    )
    def k(t_hbm, adst_hbm, sidx_hbm, didx_hbm, acc_hbm,
          sidx_v, didx_v, rows_v, gd_v, acc_sh, sem):
        cid = lax.axis_index("c")
        sid = lax.axis_index("s")
        wid = cid * 16 + sid

        pltpu.sync_copy(sidx_hbm.at[wid], sidx_v)
        pltpu.sync_copy(didx_hbm.at[wid], didx_v)

        zero16 = jnp.zeros((16,), jnp.float32)

        def zrow(r, c):
            for kk in range(_DF // 16):
                rows_v[r, pl.ds(kk * 16, 16)] = zero16
            return c

        lax.fori_loop(0, _RZ, zrow, 0)
        rbase = sid * _RPT
        for j in range(_RPT // _RZ):
            pltpu.sync_copy(rows_v.at[pl.ds(0, _RZ)],
                            acc_sh.at[pl.ds(rbase + j * _RZ, _RZ)])
        plsc.subcore_barrier()

        def block(i, c):
            pltpu.async_copy(t_hbm.at[sidx_v.at[i]], rows_v, sem).wait()
            pltpu.async_copy(adst_hbm.at[didx_v.at[i]], gd_v, sem).wait()

            def edge(e, c2):
                a = rows_v[e, pl.ds(_D, 16)] + gd_v[e, :]
                a = jnp.where(a >= 0.0, a, a * 0.2)
                w = jnp.exp(a)
                rows_v[e, pl.ds(_D, 16)] = w
                ws = None
                for kk in range(8):
                    if kk == 0 or maps[kk] != maps[kk - 1]:
                        ws = jnp.broadcast_to(w[maps[kk]], (16,))
                    sl = pl.ds(kk * 16, 16)
                    rows_v[e, sl] = rows_v[e, sl] * ws
                return c2

            lax.fori_loop(0, _B, edge, 0)
            pltpu.sync_copy(rows_v, acc_sh.at[didx_v.at[i]], add=True)
            return c

        lax.fori_loop(0, _NB, block, 0)
        plsc.subcore_barrier()

        for j in range(_RPT // _RZ):
            r0 = rbase + j * _RZ
            pltpu.sync_copy(acc_sh.at[pl.ds(r0, _RZ)],
                            acc_hbm.at[cid, pl.ds(r0, _RZ)])

    return k


def _mid_body(x_ref, t_ref, ad_ref, aa_ref, ab_ref,
              b0_ref, g_ref, be_ref, w1_ref, s1_ref, d1_ref, p_ref,
              t1_out, ad1_out):
    as16 = t_ref[:, _D:]
    a16 = as16 + ad_ref[:]
    wself = jnp.exp(jnp.where(a16 >= 0.0, a16, a16 * 0.2))
    den16 = aa_ref[:, _D:] + ab_ref[:, _D:] + wself
    wexp = jnp.dot(wself, p_ref[:], preferred_element_type=jnp.float32)
    dexp = jnp.dot(den16, p_ref[:], preferred_element_type=jnp.float32)
    num = aa_ref[:, :_D] + ab_ref[:, :_D] + wexp * t_ref[:, :_D]
    g = num / dexp + b0_ref[:]
    mu = jnp.mean(g, axis=1, keepdims=True)
    var = jnp.mean((g - mu) ** 2, axis=1, keepdims=True)
    ln = (g - mu) / jnp.sqrt(var + 1e-5) * g_ref[:] + be_ref[:]
    el = jnp.where(ln > 0.0, ln, jnp.exp(ln) - 1.0)
    h = el + x_ref[:]
    xl1 = jnp.dot(h, w1_ref[:], preferred_element_type=jnp.float32)
    t1_out[:, :_D] = xl1
    t1_out[:, _D:] = jnp.dot(xl1, s1_ref[:], preferred_element_type=jnp.float32)
    ad1_out[:] = jnp.dot(xl1, d1_ref[:], preferred_element_type=jnp.float32)


def _mid(x, t0, ad0, aa, ab, b0, g, be, W1, S1, D1, P):
    grid = (_N // _RB,)
    rb = lambda i: (i, 0)
    z = lambda i: (0, 0)
    return pl.pallas_call(
        _mid_body,
        grid=grid,
        in_specs=[
            pl.BlockSpec((_RB, _D), rb), pl.BlockSpec((_RB, _DF), rb),
            pl.BlockSpec((_RB, 16), rb),
            pl.BlockSpec((_RB, _DF), rb), pl.BlockSpec((_RB, _DF), rb),
            pl.BlockSpec((1, _D), z), pl.BlockSpec((1, _D), z),
            pl.BlockSpec((1, _D), z),
            pl.BlockSpec((_D, _D), z),
            pl.BlockSpec((_D, 16), z), pl.BlockSpec((_D, 16), z),
            pl.BlockSpec((16, _D), z),
        ],
        out_specs=[
            pl.BlockSpec((_RB, _DF), rb),
            pl.BlockSpec((_RB, 16), rb),
        ],
        out_shape=[
            jax.ShapeDtypeStruct((_N, _DF), jnp.float32),
            jax.ShapeDtypeStruct((_N, 16), jnp.float32),
        ],
    )(x, t0, ad0, aa, ab, b0, g, be, W1, S1, D1, P)


def _fin_body(t1_ref, ad_ref, aa_ref, ab_ref, b1_ref, pf_ref, out_ref):
    a16 = t1_ref[:, _D:] + ad_ref[:]
    wself = jnp.exp(jnp.where(a16 >= 0.0, a16, a16 * 0.2))
    den16 = aa_ref[:, _D:] + ab_ref[:, _D:] + wself
    w128 = jnp.dot(wself, pf_ref[:], preferred_element_type=jnp.float32)
    d128 = jnp.dot(den16, pf_ref[:], preferred_element_type=jnp.float32)
    num = aa_ref[:, :_D] + ab_ref[:, :_D] + w128 * t1_ref[:, :_D]
    out_ref[:] = num / d128 + b1_ref[:]


def _fin(t1, ad1, aa, ab, b1, PF):
    grid = (_N // _RB,)
    rb = lambda i: (i, 0)
    z = lambda i: (0, 0)
    return pl.pallas_call(
        _fin_body,
        grid=grid,
        in_specs=[
            pl.BlockSpec((_RB, _DF), rb), pl.BlockSpec((_RB, 16), rb),
            pl.BlockSpec((_RB, _DF), rb), pl.BlockSpec((_RB, _DF), rb),
            pl.BlockSpec((1, _D), z), pl.BlockSpec((16, _D), z),
        ],
        out_specs=pl.BlockSpec((_RB, _D), rb),
        out_shape=jax.ShapeDtypeStruct((_N, _D), jnp.float32),
    )(t1, ad1, aa, ab, b1, PF)


def kernel(x, edge_index, W0, att_src0, att_dst0, bias0, ln_g, ln_b,
           W1, att_src1, att_dst1, bias1):
    ei = edge_index.astype(jnp.int32)
    sidx = ei[0].reshape(_NW, _NB, _B)
    didx = ei[1].reshape(_NW, _NB, _B)

    eye8 = jnp.eye(8, 16, dtype=jnp.float32)
    S0 = (att_src0.reshape(8, 16)[:, :, None] * eye8[:, None, :]).reshape(128, 16)
    D0 = (att_dst0.reshape(8, 16)[:, :, None] * eye8[:, None, :]).reshape(128, 16)
    S1 = jnp.pad(att_src1.reshape(128, 1), ((0, 0), (0, 15)))
    D1 = jnp.pad(att_dst1.reshape(128, 1), ((0, 0), (0, 15)))
    P = jnp.repeat(jnp.eye(16, dtype=jnp.float32)[:, :8], 16, axis=1)
    PF = jnp.zeros((16, _D), jnp.float32).at[0].set(1.0)

    b0 = bias0.reshape(1, _D)
    b1 = bias1.reshape(1, _D)
    g = ln_g.reshape(1, _D)
    be = ln_b.reshape(1, _D)

    t0, ad0 = _prep(x, W0, S0, D0)
    acc0 = _make_edge_pass((0, 1, 2, 3, 4, 5, 6, 7))(t0, ad0, sidx, didx)
    t1, ad1 = _mid(x, t0, ad0, acc0[0, :_N], acc0[1, :_N],
                   b0, g, be, W1, S1, D1, P)
    acc1 = _make_edge_pass((0, 0, 0, 0, 0, 0, 0, 0))(t1, ad1, sidx, didx)
    return _fin(t1, ad1, acc1[0, :_N], acc1[1, :_N], b1, PF)


# speedup vs baseline: 57.6365x; 1.1942x over previous
import functools

import jax
import jax.numpy as jnp
from jax import lax
from jax.experimental import pallas as pl
from jax.experimental.pallas import tpu as pltpu
from jax.experimental.pallas import tpu_sc as plsc

_N = 10000
_D = 128
_E = 320000
_RB = 2000
_NW = 32
_EPW = _E // _NW
_B = 80
_NB = _EPW // _B
_DF = _D + 16
_NACC = 10240
_RPT = _NACC // 16
_RZ = 160


def _prep_body(x_ref, w_ref, s_ref, d_ref, t_out, ad_out):
    xl = jnp.dot(x_ref[:], w_ref[:], preferred_element_type=jnp.float32)
    t_out[:, :_D] = xl
    t_out[:, _D:] = jnp.dot(xl, s_ref[:], preferred_element_type=jnp.float32)
    ad_out[:] = jnp.dot(xl, d_ref[:], preferred_element_type=jnp.float32)


def _prep(x, W, S, Dm):
    grid = (_N // _RB,)
    rb = lambda i: (i, 0)
    z = lambda i: (0, 0)
    return pl.pallas_call(
        _prep_body,
        grid=grid,
        in_specs=[
            pl.BlockSpec((_RB, _D), rb),
            pl.BlockSpec((_D, _D), z),
            pl.BlockSpec((_D, 16), z),
            pl.BlockSpec((_D, 16), z),
        ],
        out_specs=[
            pl.BlockSpec((_RB, _DF), rb),
            pl.BlockSpec((_RB, 16), rb),
        ],
        out_shape=[
            jax.ShapeDtypeStruct((_N, _DF), jnp.float32),
            jax.ShapeDtypeStruct((_N, 16), jnp.float32),
        ],
    )(x, W, S, Dm)


@functools.lru_cache(maxsize=None)
def _make_edge_pass(maps):
    mesh = plsc.VectorSubcoreMesh(core_axis_name="c", subcore_axis_name="s")

    @functools.partial(
        pl.kernel,
        mesh=mesh,
        compiler_params=pltpu.CompilerParams(use_tc_tiling_on_sc=False),
        out_type=jax.ShapeDtypeStruct((2, _NACC, _DF), jnp.float32),
        scratch_types=[
            pltpu.VMEM((2, _B), jnp.int32),
            pltpu.VMEM((2, _B), jnp.int32),
            pltpu.VMEM((2, _B, _DF), jnp.float32),
            pltpu.VMEM((2, _B, 16), jnp.float32),
            pltpu.VMEM_SHARED((_NACC, _DF), jnp.float32),
            pltpu.SemaphoreType.DMA,
            pltpu.SemaphoreType.DMA,
        ],
    )
    def k(t_hbm, adst_hbm, sidx_hbm, didx_hbm, acc_hbm,
          sidx_v, didx_v, rows_v, gd_v, acc_sh, sg0, sg1):
        cid = lax.axis_index("c")
        sid = lax.axis_index("s")
        wid = cid * 16 + sid
        sg = (sg0, sg1)

        zero16 = jnp.zeros((16,), jnp.float32)

        def zrow(r, c):
            for kk in range(_DF // 16):
                rows_v[0, r, pl.ds(kk * 16, 16)] = zero16
            return c

        lax.fori_loop(0, _B, zrow, 0)
        rbase = sid * _RPT
        for j in range(_RPT // _B):
            pltpu.sync_copy(rows_v.at[0],
                            acc_sh.at[pl.ds(rbase + j * _B, _B)])
        plsc.subcore_barrier()

        def stage_and_start(i, slot):
            pltpu.sync_copy(sidx_hbm.at[wid, i], sidx_v.at[slot])
            pltpu.sync_copy(didx_hbm.at[wid, i], didx_v.at[slot])
            pltpu.async_copy(t_hbm.at[sidx_v.at[slot]], rows_v.at[slot],
                             sg[slot])
            pltpu.async_copy(adst_hbm.at[didx_v.at[slot]], gd_v.at[slot],
                             sg[slot])

        def wait_gathers(slot):
            pltpu.make_async_copy(t_hbm.at[sidx_v.at[slot]],
                                  rows_v.at[slot], sg[slot]).wait()
            pltpu.make_async_copy(adst_hbm.at[didx_v.at[slot]],
                                  gd_v.at[slot], sg[slot]).wait()

        def compute_and_scatter(slot):
            def edge(e, c2):
                a = rows_v[slot, e, pl.ds(_D, 16)] + gd_v[slot, e, :]
                a = jnp.where(a >= 0.0, a, a * 0.2)
                w = jnp.exp(a)
                rows_v[slot, e, pl.ds(_D, 16)] = w
                ws = None
                for kk in range(8):
                    if kk == 0 or maps[kk] != maps[kk - 1]:
                        ws = jnp.broadcast_to(w[maps[kk]], (16,))
                    sl = pl.ds(kk * 16, 16)
                    rows_v[slot, e, sl] = rows_v[slot, e, sl] * ws
                return c2

            lax.fori_loop(0, _B, edge, 0)
            pltpu.sync_copy(rows_v.at[slot], acc_sh.at[didx_v.at[slot]],
                            add=True)

        stage_and_start(0, 0)

        def body(j, c):
            i = j * 2
            stage_and_start(i + 1, 1)
            wait_gathers(0)
            compute_and_scatter(0)
            stage_and_start(i + 2, 0)
            wait_gathers(1)
            compute_and_scatter(1)
            return c

        lax.fori_loop(0, (_NB - 1) // 2, body, 0)
        wait_gathers(0)
        compute_and_scatter(0)
        plsc.subcore_barrier()

        for j in range(_RPT // _RZ):
            r0 = rbase + j * _RZ
            pltpu.sync_copy(acc_sh.at[pl.ds(r0, _RZ)],
                            acc_hbm.at[cid, pl.ds(r0, _RZ)])

    return k


def _mid_body(x_ref, t_ref, ad_ref, aa_ref, ab_ref,
              b0_ref, g_ref, be_ref, w1_ref, s1_ref, d1_ref, p_ref,
              t1_out, ad1_out):
    as16 = t_ref[:, _D:]
    a16 = as16 + ad_ref[:]
    wself = jnp.exp(jnp.where(a16 >= 0.0, a16, a16 * 0.2))
    den16 = aa_ref[:, _D:] + ab_ref[:, _D:] + wself
    wexp = jnp.dot(wself, p_ref[:], preferred_element_type=jnp.float32)
    dexp = jnp.dot(den16, p_ref[:], preferred_element_type=jnp.float32)
    num = aa_ref[:, :_D] + ab_ref[:, :_D] + wexp * t_ref[:, :_D]
    g = num / dexp + b0_ref[:]
    mu = jnp.mean(g, axis=1, keepdims=True)
    var = jnp.mean((g - mu) ** 2, axis=1, keepdims=True)
    ln = (g - mu) / jnp.sqrt(var + 1e-5) * g_ref[:] + be_ref[:]
    el = jnp.where(ln > 0.0, ln, jnp.exp(ln) - 1.0)
    h = el + x_ref[:]
    xl1 = jnp.dot(h, w1_ref[:], preferred_element_type=jnp.float32)
    t1_out[:, :_D] = xl1
    t1_out[:, _D:] = jnp.dot(xl1, s1_ref[:], preferred_element_type=jnp.float32)
    ad1_out[:] = jnp.dot(xl1, d1_ref[:], preferred_element_type=jnp.float32)


def _mid(x, t0, ad0, aa, ab, b0, g, be, W1, S1, D1, P):
    grid = (_N // _RB,)
    rb = lambda i: (i, 0)
    z = lambda i: (0, 0)
    return pl.pallas_call(
        _mid_body,
        grid=grid,
        in_specs=[
            pl.BlockSpec((_RB, _D), rb), pl.BlockSpec((_RB, _DF), rb),
            pl.BlockSpec((_RB, 16), rb),
            pl.BlockSpec((_RB, _DF), rb), pl.BlockSpec((_RB, _DF), rb),
            pl.BlockSpec((1, _D), z), pl.BlockSpec((1, _D), z),
            pl.BlockSpec((1, _D), z),
            pl.BlockSpec((_D, _D), z),
            pl.BlockSpec((_D, 16), z), pl.BlockSpec((_D, 16), z),
            pl.BlockSpec((16, _D), z),
        ],
        out_specs=[
            pl.BlockSpec((_RB, _DF), rb),
            pl.BlockSpec((_RB, 16), rb),
        ],
        out_shape=[
            jax.ShapeDtypeStruct((_N, _DF), jnp.float32),
            jax.ShapeDtypeStruct((_N, 16), jnp.float32),
        ],
    )(x, t0, ad0, aa, ab, b0, g, be, W1, S1, D1, P)


def _fin_body(t1_ref, ad_ref, aa_ref, ab_ref, b1_ref, pf_ref, out_ref):
    a16 = t1_ref[:, _D:] + ad_ref[:]
    wself = jnp.exp(jnp.where(a16 >= 0.0, a16, a16 * 0.2))
    den16 = aa_ref[:, _D:] + ab_ref[:, _D:] + wself
    w128 = jnp.dot(wself, pf_ref[:], preferred_element_type=jnp.float32)
    d128 = jnp.dot(den16, pf_ref[:], preferred_element_type=jnp.float32)
    num = aa_ref[:, :_D] + ab_ref[:, :_D] + w128 * t1_ref[:, :_D]
    out_ref[:] = num / d128 + b1_ref[:]


def _fin(t1, ad1, aa, ab, b1, PF):
    grid = (_N // _RB,)
    rb = lambda i: (i, 0)
    z = lambda i: (0, 0)
    return pl.pallas_call(
        _fin_body,
        grid=grid,
        in_specs=[
            pl.BlockSpec((_RB, _DF), rb), pl.BlockSpec((_RB, 16), rb),
            pl.BlockSpec((_RB, _DF), rb), pl.BlockSpec((_RB, _DF), rb),
            pl.BlockSpec((1, _D), z), pl.BlockSpec((16, _D), z),
        ],
        out_specs=pl.BlockSpec((_RB, _D), rb),
        out_shape=jax.ShapeDtypeStruct((_N, _D), jnp.float32),
    )(t1, ad1, aa, ab, b1, PF)


def kernel(x, edge_index, W0, att_src0, att_dst0, bias0, ln_g, ln_b,
           W1, att_src1, att_dst1, bias1):
    ei = edge_index.astype(jnp.int32)
    sidx = ei[0].reshape(_NW, _NB, _B)
    didx = ei[1].reshape(_NW, _NB, _B)

    eye8 = jnp.eye(8, 16, dtype=jnp.float32)
    S0 = (att_src0.reshape(8, 16)[:, :, None] * eye8[:, None, :]).reshape(128, 16)
    D0 = (att_dst0.reshape(8, 16)[:, :, None] * eye8[:, None, :]).reshape(128, 16)
    S1 = jnp.pad(att_src1.reshape(128, 1), ((0, 0), (0, 15)))
    D1 = jnp.pad(att_dst1.reshape(128, 1), ((0, 0), (0, 15)))
    P = jnp.repeat(jnp.eye(16, dtype=jnp.float32)[:, :8], 16, axis=1)
    PF = jnp.zeros((16, _D), jnp.float32).at[0].set(1.0)

    b0 = bias0.reshape(1, _D)
    b1 = bias1.reshape(1, _D)
    g = ln_g.reshape(1, _D)
    be = ln_b.reshape(1, _D)

    t0, ad0 = _prep(x, W0, S0, D0)
    acc0 = _make_edge_pass((0, 1, 2, 3, 4, 5, 6, 7))(t0, ad0, sidx, didx)
    t1, ad1 = _mid(x, t0, ad0, acc0[0, :_N], acc0[1, :_N],
                   b0, g, be, W1, S1, D1, P)
    acc1 = _make_edge_pass((0, 0, 0, 0, 0, 0, 0, 0))(t1, ad1, sidx, didx)
    return _fin(t1, ad1, acc1[0, :_N], acc1[1, :_N], b1, PF)


# speedup vs baseline: 65.8123x; 1.1419x over previous
import functools

import jax
import jax.numpy as jnp
from jax import lax
from jax.experimental import pallas as pl
from jax.experimental.pallas import tpu as pltpu
from jax.experimental.pallas import tpu_sc as plsc

_N = 10000
_D = 128
_E = 320000
_RB = 2000
_NW = 32
_EPW = _E // _NW
_B = 80
_NB = _EPW // _B
_DF = _D + 16
_NACC = 10112
_RPT = _NACC // 16


def _prep_body(x_ref, w_ref, s_ref, d_ref, t_out, ad_out):
    xl = jnp.dot(x_ref[:], w_ref[:], preferred_element_type=jnp.float32)
    t_out[:, :_D] = xl
    t_out[:, _D:] = jnp.dot(xl, s_ref[:], preferred_element_type=jnp.float32)
    ad_out[:] = jnp.dot(xl, d_ref[:], preferred_element_type=jnp.float32)


def _prep(x, W, S, Dm):
    grid = (_N // _RB,)
    rb = lambda i: (i, 0)
    z = lambda i: (0, 0)
    return pl.pallas_call(
        _prep_body,
        grid=grid,
        in_specs=[
            pl.BlockSpec((_RB, _D), rb),
            pl.BlockSpec((_D, _D), z),
            pl.BlockSpec((_D, 16), z),
            pl.BlockSpec((_D, 16), z),
        ],
        out_specs=[
            pl.BlockSpec((_RB, _DF), rb),
            pl.BlockSpec((_RB, 16), rb),
        ],
        out_shape=[
            jax.ShapeDtypeStruct((_N, _DF), jnp.float32),
            jax.ShapeDtypeStruct((_N, 16), jnp.float32),
        ],
    )(x, W, S, Dm)


@functools.lru_cache(maxsize=None)
def _make_edge_pass(maps):
    mesh = plsc.VectorSubcoreMesh(core_axis_name="c", subcore_axis_name="s")

    @functools.partial(
        pl.kernel,
        mesh=mesh,
        compiler_params=pltpu.CompilerParams(use_tc_tiling_on_sc=False),
        out_type=jax.ShapeDtypeStruct((2, _NACC, _DF), jnp.float32),
        scratch_types=[
            pltpu.VMEM((3, _B), jnp.int32),
            pltpu.VMEM((3, _B), jnp.int32),
            pltpu.VMEM((3, _B, _DF), jnp.float32),
            pltpu.VMEM((3, _B, 16), jnp.float32),
            pltpu.VMEM_SHARED((_NACC, _DF), jnp.float32),
            pltpu.SemaphoreType.DMA,
            pltpu.SemaphoreType.DMA,
            pltpu.SemaphoreType.DMA,
            pltpu.SemaphoreType.DMA,
            pltpu.SemaphoreType.DMA,
            pltpu.SemaphoreType.DMA,
        ],
    )
    def k(t_hbm, adst_hbm, sidx_hbm, didx_hbm, acc_hbm,
          sidx_v, didx_v, rows_v, gd_v, acc_sh,
          sg0, sg1, sg2, ss0, ss1, ss2):
        cid = lax.axis_index("c")
        sid = lax.axis_index("s")
        wid = cid * 16 + sid
        sg = (sg0, sg1, sg2)
        ss = (ss0, ss1, ss2)

        zero16 = jnp.zeros((16,), jnp.float32)

        def zrow(r, c):
            for kk in range(_DF // 16):
                rows_v[0, r, pl.ds(kk * 16, 16)] = zero16
            return c

        lax.fori_loop(0, _B, zrow, 0)
        rbase = sid * _RPT
        for j in range(7):
            pltpu.sync_copy(rows_v.at[0],
                            acc_sh.at[pl.ds(rbase + j * _B, _B)])
        pltpu.sync_copy(rows_v.at[0, pl.ds(0, _RPT - 7 * _B)],
                        acc_sh.at[pl.ds(rbase + 7 * _B, _RPT - 7 * _B)])
        plsc.subcore_barrier()

        def stage_and_start(i, slot):
            pltpu.sync_copy(sidx_hbm.at[wid, i], sidx_v.at[slot])
            pltpu.sync_copy(didx_hbm.at[wid, i], didx_v.at[slot])
            pltpu.async_copy(t_hbm.at[sidx_v.at[slot]], rows_v.at[slot],
                             sg[slot])
            pltpu.async_copy(adst_hbm.at[didx_v.at[slot]], gd_v.at[slot],
                             sg[slot])

        def wait_gathers(slot):
            pltpu.make_async_copy(t_hbm.at[sidx_v.at[slot]],
                                  rows_v.at[slot], sg[slot]).wait()
            pltpu.make_async_copy(adst_hbm.at[didx_v.at[slot]],
                                  gd_v.at[slot], sg[slot]).wait()

        def drain_scatter(slot):
            pltpu.make_async_copy(rows_v.at[slot],
                                  acc_sh.at[didx_v.at[slot]],
                                  ss[slot]).wait()

        def compute(slot):
            def edge(e, c2):
                a = rows_v[slot, e, pl.ds(_D, 16)] + gd_v[slot, e, :]
                a = jnp.where(a >= 0.0, a, a * 0.2)
                w = jnp.exp(a)
                rows_v[slot, e, pl.ds(_D, 16)] = w
                ws = None
                for kk in range(8):
                    if kk == 0 or maps[kk] != maps[kk - 1]:
                        ws = jnp.broadcast_to(w[maps[kk]], (16,))
                    sl = pl.ds(kk * 16, 16)
                    rows_v[slot, e, sl] = rows_v[slot, e, sl] * ws
                return c2

            lax.fori_loop(0, _B, edge, 0)
            pltpu.async_copy(rows_v.at[slot], acc_sh.at[didx_v.at[slot]],
                             ss[slot], add=True)

        stage_and_start(0, 0)
        stage_and_start(1, 1)
        wait_gathers(0)
        compute(0)
        stage_and_start(2, 2)
        wait_gathers(1)
        compute(1)

        def sec(i, slot):
            nslot = (slot + 1) % 3
            drain_scatter(nslot)
            stage_and_start(i + 1, nslot)
            wait_gathers(slot)
            compute(slot)

        def body(j, c):
            i = j * 3 + 2
            sec(i, 2)
            sec(i + 1, 0)
            sec(i + 2, 1)
            return c

        lax.fori_loop(0, (_NB - 5) // 3, body, 0)
        sec(_NB - 3, (_NB - 3) % 3)
        sec(_NB - 2, (_NB - 2) % 3)
        lastslot = (_NB - 1) % 3
        wait_gathers(lastslot)
        compute(lastslot)
        for s in range(3):
            drain_scatter(s)
        plsc.subcore_barrier()

        for j in range(7):
            r0 = rbase + j * _B
            pltpu.sync_copy(acc_sh.at[pl.ds(r0, _B)],
                            acc_hbm.at[cid, pl.ds(r0, _B)])
        r0 = rbase + 7 * _B
        pltpu.sync_copy(acc_sh.at[pl.ds(r0, _RPT - 7 * _B)],
                        acc_hbm.at[cid, pl.ds(r0, _RPT - 7 * _B)])

    return k


def _mid_body(x_ref, t_ref, ad_ref, aa_ref, ab_ref,
              b0_ref, g_ref, be_ref, w1_ref, s1_ref, d1_ref, p_ref,
              t1_out, ad1_out):
    as16 = t_ref[:, _D:]
    a16 = as16 + ad_ref[:]
    wself = jnp.exp(jnp.where(a16 >= 0.0, a16, a16 * 0.2))
    den16 = aa_ref[:, _D:] + ab_ref[:, _D:] + wself
    wexp = jnp.dot(wself, p_ref[:], preferred_element_type=jnp.float32)
    dexp = jnp.dot(den16, p_ref[:], preferred_element_type=jnp.float32)
    num = aa_ref[:, :_D] + ab_ref[:, :_D] + wexp * t_ref[:, :_D]
    g = num / dexp + b0_ref[:]
    mu = jnp.mean(g, axis=1, keepdims=True)
    var = jnp.mean((g - mu) ** 2, axis=1, keepdims=True)
    ln = (g - mu) / jnp.sqrt(var + 1e-5) * g_ref[:] + be_ref[:]
    el = jnp.where(ln > 0.0, ln, jnp.exp(ln) - 1.0)
    h = el + x_ref[:]
    xl1 = jnp.dot(h, w1_ref[:], preferred_element_type=jnp.float32)
    t1_out[:, :_D] = xl1
    t1_out[:, _D:] = jnp.dot(xl1, s1_ref[:], preferred_element_type=jnp.float32)
    ad1_out[:] = jnp.dot(xl1, d1_ref[:], preferred_element_type=jnp.float32)


def _mid(x, t0, ad0, aa, ab, b0, g, be, W1, S1, D1, P):
    grid = (_N // _RB,)
    rb = lambda i: (i, 0)
    z = lambda i: (0, 0)
    return pl.pallas_call(
        _mid_body,
        grid=grid,
        in_specs=[
            pl.BlockSpec((_RB, _D), rb), pl.BlockSpec((_RB, _DF), rb),
            pl.BlockSpec((_RB, 16), rb),
            pl.BlockSpec((_RB, _DF), rb), pl.BlockSpec((_RB, _DF), rb),
            pl.BlockSpec((1, _D), z), pl.BlockSpec((1, _D), z),
            pl.BlockSpec((1, _D), z),
            pl.BlockSpec((_D, _D), z),
            pl.BlockSpec((_D, 16), z), pl.BlockSpec((_D, 16), z),
            pl.BlockSpec((16, _D), z),
        ],
        out_specs=[
            pl.BlockSpec((_RB, _DF), rb),
            pl.BlockSpec((_RB, 16), rb),
        ],
        out_shape=[
            jax.ShapeDtypeStruct((_N, _DF), jnp.float32),
            jax.ShapeDtypeStruct((_N, 16), jnp.float32),
        ],
    )(x, t0, ad0, aa, ab, b0, g, be, W1, S1, D1, P)


def _fin_body(t1_ref, ad_ref, aa_ref, ab_ref, b1_ref, pf_ref, out_ref):
    a16 = t1_ref[:, _D:] + ad_ref[:]
    wself = jnp.exp(jnp.where(a16 >= 0.0, a16, a16 * 0.2))
    den16 = aa_ref[:, _D:] + ab_ref[:, _D:] + wself
    w128 = jnp.dot(wself, pf_ref[:], preferred_element_type=jnp.float32)
    d128 = jnp.dot(den16, pf_ref[:], preferred_element_type=jnp.float32)
    num = aa_ref[:, :_D] + ab_ref[:, :_D] + w128 * t1_ref[:, :_D]
    out_ref[:] = num / d128 + b1_ref[:]


def _fin(t1, ad1, aa, ab, b1, PF):
    grid = (_N // _RB,)
    rb = lambda i: (i, 0)
    z = lambda i: (0, 0)
    return pl.pallas_call(
        _fin_body,
        grid=grid,
        in_specs=[
            pl.BlockSpec((_RB, _DF), rb), pl.BlockSpec((_RB, 16), rb),
            pl.BlockSpec((_RB, _DF), rb), pl.BlockSpec((_RB, _DF), rb),
            pl.BlockSpec((1, _D), z), pl.BlockSpec((16, _D), z),
        ],
        out_specs=pl.BlockSpec((_RB, _D), rb),
        out_shape=jax.ShapeDtypeStruct((_N, _D), jnp.float32),
    )(t1, ad1, aa, ab, b1, PF)


def kernel(x, edge_index, W0, att_src0, att_dst0, bias0, ln_g, ln_b,
           W1, att_src1, att_dst1, bias1):
    ei = edge_index.astype(jnp.int32)
    sidx = ei[0].reshape(_NW, _NB, _B)
    didx = ei[1].reshape(_NW, _NB, _B)

    eye8 = jnp.eye(8, 16, dtype=jnp.float32)
    S0 = (att_src0.reshape(8, 16)[:, :, None] * eye8[:, None, :]).reshape(128, 16)
    D0 = (att_dst0.reshape(8, 16)[:, :, None] * eye8[:, None, :]).reshape(128, 16)
    S1 = jnp.pad(att_src1.reshape(128, 1), ((0, 0), (0, 15)))
    D1 = jnp.pad(att_dst1.reshape(128, 1), ((0, 0), (0, 15)))
    P = jnp.repeat(jnp.eye(16, dtype=jnp.float32)[:, :8], 16, axis=1)
    PF = jnp.zeros((16, _D), jnp.float32).at[0].set(1.0)

    b0 = bias0.reshape(1, _D)
    b1 = bias1.reshape(1, _D)
    g = ln_g.reshape(1, _D)
    be = ln_b.reshape(1, _D)

    t0, ad0 = _prep(x, W0, S0, D0)
    acc0 = _make_edge_pass((0, 1, 2, 3, 4, 5, 6, 7))(t0, ad0, sidx, didx)
    t1, ad1 = _mid(x, t0, ad0, acc0[0, :_N], acc0[1, :_N],
                   b0, g, be, W1, S1, D1, P)
    acc1 = _make_edge_pass((0, 0, 0, 0, 0, 0, 0, 0))(t1, ad1, sidx, didx)
    return _fin(t1, ad1, acc1[0, :_N], acc1[1, :_N], b1, PF)


# speedup vs baseline: 67.1821x; 1.0208x over previous
import functools

import jax
import jax.numpy as jnp
from jax import lax
from jax.experimental import pallas as pl
from jax.experimental.pallas import tpu as pltpu
from jax.experimental.pallas import tpu_sc as plsc

_N = 10000
_D = 128
_E = 320000
_RB = 2000
_NW = 32
_EPW = _E // _NW
_B = 80
_NB = _EPW // _B
_DF = _D + 16
_NACC = 10112
_RPT = _NACC // 16


def _prep_body(x_ref, w_ref, s_ref, d_ref, t_out, ad_out):
    xl = jnp.dot(x_ref[:], w_ref[:], preferred_element_type=jnp.float32)
    t_out[:, :_D] = xl
    t_out[:, _D:] = jnp.dot(xl, s_ref[:], preferred_element_type=jnp.float32)
    ad_out[:] = jnp.dot(xl, d_ref[:], preferred_element_type=jnp.float32)


def _prep(x, W, S, Dm):
    grid = (_N // _RB,)
    rb = lambda i: (i, 0)
    z = lambda i: (0, 0)
    return pl.pallas_call(
        _prep_body,
        grid=grid,
        in_specs=[
            pl.BlockSpec((_RB, _D), rb),
            pl.BlockSpec((_D, _D), z),
            pl.BlockSpec((_D, 16), z),
            pl.BlockSpec((_D, 16), z),
        ],
        out_specs=[
            pl.BlockSpec((_RB, _DF), rb),
            pl.BlockSpec((_RB, 16), rb),
        ],
        out_shape=[
            jax.ShapeDtypeStruct((_N, _DF), jnp.float32),
            jax.ShapeDtypeStruct((_N, 16), jnp.float32),
        ],
    )(x, W, S, Dm)


@functools.lru_cache(maxsize=None)
def _make_edge_pass(maps):
    mesh = plsc.VectorSubcoreMesh(core_axis_name="c", subcore_axis_name="s")

    @functools.partial(
        pl.kernel,
        mesh=mesh,
        compiler_params=pltpu.CompilerParams(use_tc_tiling_on_sc=False),
        out_type=jax.ShapeDtypeStruct((2, _NACC, _DF), jnp.float32),
        scratch_types=[
            pltpu.VMEM((3, _B), jnp.int32),
            pltpu.VMEM((3, _B), jnp.int32),
            pltpu.VMEM((3, _B, _DF), jnp.float32),
            pltpu.VMEM((3, _B, 16), jnp.float32),
            pltpu.VMEM_SHARED((_NACC, _DF), jnp.float32),
            pltpu.SemaphoreType.DMA,
            pltpu.SemaphoreType.DMA,
            pltpu.SemaphoreType.DMA,
            pltpu.SemaphoreType.DMA,
            pltpu.SemaphoreType.DMA,
            pltpu.SemaphoreType.DMA,
        ],
    )
    def k(t_hbm, adst_hbm, sidx_hbm, didx_hbm, acc_hbm,
          sidx_v, didx_v, rows_v, gd_v, acc_sh,
          sg0, sg1, sg2, ss0, ss1, ss2):
        cid = lax.axis_index("c")
        sid = lax.axis_index("s")
        wid = cid * 16 + sid
        sg = (sg0, sg1, sg2)
        ss = (ss0, ss1, ss2)

        zero16 = jnp.zeros((16,), jnp.float32)

        def zrow(r, c):
            for kk in range(_DF // 16):
                rows_v[0, r, pl.ds(kk * 16, 16)] = zero16
            return c

        lax.fori_loop(0, _B, zrow, 0)
        rbase = sid * _RPT
        for j in range(7):
            pltpu.sync_copy(rows_v.at[0],
                            acc_sh.at[pl.ds(rbase + j * _B, _B)])
        pltpu.sync_copy(rows_v.at[0, pl.ds(0, _RPT - 7 * _B)],
                        acc_sh.at[pl.ds(rbase + 7 * _B, _RPT - 7 * _B)])
        plsc.subcore_barrier()

        def stage_and_start(i, slot):
            pltpu.sync_copy(sidx_hbm.at[wid, i], sidx_v.at[slot])
            pltpu.sync_copy(didx_hbm.at[wid, i], didx_v.at[slot])
            pltpu.async_copy(t_hbm.at[sidx_v.at[slot]], rows_v.at[slot],
                             sg[slot])
            pltpu.async_copy(adst_hbm.at[didx_v.at[slot]], gd_v.at[slot],
                             sg[slot])

        def wait_gathers(slot):
            pltpu.make_async_copy(t_hbm.at[sidx_v.at[slot]],
                                  rows_v.at[slot], sg[slot]).wait()
            pltpu.make_async_copy(adst_hbm.at[didx_v.at[slot]],
                                  gd_v.at[slot], sg[slot]).wait()

        def drain_scatter(slot):
            pltpu.make_async_copy(rows_v.at[slot],
                                  acc_sh.at[didx_v.at[slot]],
                                  ss[slot]).wait()

        def compute(slot):
            def edge(e, c2):
                a = rows_v[slot, e, pl.ds(_D, 16)] + gd_v[slot, e, :]
                a = jnp.where(a >= 0.0, a, a * 0.2)
                w = jnp.exp(a)
                rows_v[slot, e, pl.ds(_D, 16)] = w
                ws = None
                for kk in range(8):
                    if kk == 0 or maps[kk] != maps[kk - 1]:
                        ws = jnp.broadcast_to(w[maps[kk]], (16,))
                    sl = pl.ds(kk * 16, 16)
                    rows_v[slot, e, sl] = rows_v[slot, e, sl] * ws
                return c2

            lax.fori_loop(0, _B, edge, 0, unroll=4)
            pltpu.async_copy(rows_v.at[slot], acc_sh.at[didx_v.at[slot]],
                             ss[slot], add=True)

        stage_and_start(0, 0)
        stage_and_start(1, 1)
        wait_gathers(0)
        compute(0)
        stage_and_start(2, 2)
        wait_gathers(1)
        compute(1)

        def sec(i, slot):
            nslot = (slot + 1) % 3
            drain_scatter(nslot)
            stage_and_start(i + 1, nslot)
            wait_gathers(slot)
            compute(slot)

        def body(j, c):
            i = j * 3 + 2
            sec(i, 2)
            sec(i + 1, 0)
            sec(i + 2, 1)
            return c

        lax.fori_loop(0, (_NB - 5) // 3, body, 0)
        sec(_NB - 3, (_NB - 3) % 3)
        sec(_NB - 2, (_NB - 2) % 3)
        lastslot = (_NB - 1) % 3
        wait_gathers(lastslot)
        compute(lastslot)
        for s in range(3):
            drain_scatter(s)
        plsc.subcore_barrier()

        for j in range(7):
            r0 = rbase + j * _B
            pltpu.sync_copy(acc_sh.at[pl.ds(r0, _B)],
                            acc_hbm.at[cid, pl.ds(r0, _B)])
        r0 = rbase + 7 * _B
        pltpu.sync_copy(acc_sh.at[pl.ds(r0, _RPT - 7 * _B)],
                        acc_hbm.at[cid, pl.ds(r0, _RPT - 7 * _B)])

    return k


def _mid_body(x_ref, t_ref, ad_ref, aa_ref, ab_ref,
              b0_ref, g_ref, be_ref, w1_ref, s1_ref, d1_ref, p_ref,
              t1_out, ad1_out):
    as16 = t_ref[:, _D:]
    a16 = as16 + ad_ref[:]
    wself = jnp.exp(jnp.where(a16 >= 0.0, a16, a16 * 0.2))
    den16 = aa_ref[0, :, _D:] + ab_ref[0, :, _D:] + wself
    wexp = jnp.dot(wself, p_ref[:], preferred_element_type=jnp.float32)
    dexp = jnp.dot(den16, p_ref[:], preferred_element_type=jnp.float32)
    num = aa_ref[0, :, :_D] + ab_ref[0, :, :_D] + wexp * t_ref[:, :_D]
    g = num / dexp + b0_ref[:]
    mu = jnp.mean(g, axis=1, keepdims=True)
    var = jnp.mean((g - mu) ** 2, axis=1, keepdims=True)
    ln = (g - mu) / jnp.sqrt(var + 1e-5) * g_ref[:] + be_ref[:]
    el = jnp.where(ln > 0.0, ln, jnp.exp(ln) - 1.0)
    h = el + x_ref[:]
    xl1 = jnp.dot(h, w1_ref[:], preferred_element_type=jnp.float32)
    t1_out[:, :_D] = xl1
    t1_out[:, _D:] = jnp.dot(xl1, s1_ref[:], preferred_element_type=jnp.float32)
    ad1_out[:] = jnp.dot(xl1, d1_ref[:], preferred_element_type=jnp.float32)


def _mid(x, t0, ad0, aa, ab, b0, g, be, W1, S1, D1, P):
    grid = (_N // _RB,)
    rb = lambda i: (i, 0)
    z = lambda i: (0, 0)
    return pl.pallas_call(
        _mid_body,
        grid=grid,
        in_specs=[
            pl.BlockSpec((_RB, _D), rb), pl.BlockSpec((_RB, _DF), rb),
            pl.BlockSpec((_RB, 16), rb),
            pl.BlockSpec((1, _RB, _DF), lambda i: (0, i, 0)),
            pl.BlockSpec((1, _RB, _DF), lambda i: (1, i, 0)),
            pl.BlockSpec((1, _D), z), pl.BlockSpec((1, _D), z),
            pl.BlockSpec((1, _D), z),
            pl.BlockSpec((_D, _D), z),
            pl.BlockSpec((_D, 16), z), pl.BlockSpec((_D, 16), z),
            pl.BlockSpec((16, _D), z),
        ],
        out_specs=[
            pl.BlockSpec((_RB, _DF), rb),
            pl.BlockSpec((_RB, 16), rb),
        ],
        out_shape=[
            jax.ShapeDtypeStruct((_N, _DF), jnp.float32),
            jax.ShapeDtypeStruct((_N, 16), jnp.float32),
        ],
    )(x, t0, ad0, aa, ab, b0, g, be, W1, S1, D1, P)


def _fin_body(t1_ref, ad_ref, aa_ref, ab_ref, b1_ref, pf_ref, out_ref):
    a16 = t1_ref[:, _D:] + ad_ref[:]
    wself = jnp.exp(jnp.where(a16 >= 0.0, a16, a16 * 0.2))
    den16 = aa_ref[0, :, _D:] + ab_ref[0, :, _D:] + wself
    w128 = jnp.dot(wself, pf_ref[:], preferred_element_type=jnp.float32)
    d128 = jnp.dot(den16, pf_ref[:], preferred_element_type=jnp.float32)
    num = aa_ref[0, :, :_D] + ab_ref[0, :, :_D] + w128 * t1_ref[:, :_D]
    out_ref[:] = num / d128 + b1_ref[:]


def _fin(t1, ad1, aa, ab, b1, PF):
    grid = (_N // _RB,)
    rb = lambda i: (i, 0)
    z = lambda i: (0, 0)
    return pl.pallas_call(
        _fin_body,
        grid=grid,
        in_specs=[
            pl.BlockSpec((_RB, _DF), rb), pl.BlockSpec((_RB, 16), rb),
            pl.BlockSpec((1, _RB, _DF), lambda i: (0, i, 0)),
            pl.BlockSpec((1, _RB, _DF), lambda i: (1, i, 0)),
            pl.BlockSpec((1, _D), z), pl.BlockSpec((16, _D), z),
        ],
        out_specs=pl.BlockSpec((_RB, _D), rb),
        out_shape=jax.ShapeDtypeStruct((_N, _D), jnp.float32),
    )(t1, ad1, aa, ab, b1, PF)


def kernel(x, edge_index, W0, att_src0, att_dst0, bias0, ln_g, ln_b,
           W1, att_src1, att_dst1, bias1):
    ei = edge_index.astype(jnp.int32)
    sidx = ei[0].reshape(_NW, _NB, _B)
    didx = ei[1].reshape(_NW, _NB, _B)

    eye8 = jnp.eye(8, 16, dtype=jnp.float32)
    S0 = (att_src0.reshape(8, 16)[:, :, None] * eye8[:, None, :]).reshape(128, 16)
    D0 = (att_dst0.reshape(8, 16)[:, :, None] * eye8[:, None, :]).reshape(128, 16)
    S1 = jnp.pad(att_src1.reshape(128, 1), ((0, 0), (0, 15)))
    D1 = jnp.pad(att_dst1.reshape(128, 1), ((0, 0), (0, 15)))
    P = jnp.repeat(jnp.eye(16, dtype=jnp.float32)[:, :8], 16, axis=1)
    PF = jnp.zeros((16, _D), jnp.float32).at[0].set(1.0)

    b0 = bias0.reshape(1, _D)
    b1 = bias1.reshape(1, _D)
    g = ln_g.reshape(1, _D)
    be = ln_b.reshape(1, _D)

    t0, ad0 = _prep(x, W0, S0, D0)
    acc0 = _make_edge_pass((0, 1, 2, 3, 4, 5, 6, 7))(t0, ad0, sidx, didx)
    t1, ad1 = _mid(x, t0, ad0, acc0, acc0, b0, g, be, W1, S1, D1, P)
    acc1 = _make_edge_pass((0, 0, 0, 0, 0, 0, 0, 0))(t1, ad1, sidx, didx)
    return _fin(t1, ad1, acc1, acc1, b1, PF)


# speedup vs baseline: 91.4104x; 1.3606x over previous
import functools

import jax
import jax.numpy as jnp
from jax import lax
from jax.experimental import pallas as pl
from jax.experimental.pallas import tpu as pltpu
from jax.experimental.pallas import tpu_sc as plsc

_N = 10000
_D = 128
_E = 320000
_RB = 2000
_NW = 32
_EPW = _E // _NW
_B = 80
_NB = _EPW // _B
_DF = _D + 16
_NACC = 10112
_RPT = _NACC // 16


def _prep_body(x_ref, w_ref, s_ref, d_ref, t_out, ad_out):
    xl = jnp.dot(x_ref[:], w_ref[:], preferred_element_type=jnp.float32)
    t_out[:, :_D] = xl
    t_out[:, _D:] = jnp.dot(xl, s_ref[:], preferred_element_type=jnp.float32)
    ad_out[:] = jnp.dot(xl, d_ref[:], preferred_element_type=jnp.float32)


def _prep(x, W, S, Dm):
    grid = (_N // _RB,)
    rb = lambda i: (i, 0)
    z = lambda i: (0, 0)
    return pl.pallas_call(
        _prep_body,
        grid=grid,
        in_specs=[
            pl.BlockSpec((_RB, _D), rb),
            pl.BlockSpec((_D, _D), z),
            pl.BlockSpec((_D, 16), z),
            pl.BlockSpec((_D, 16), z),
        ],
        out_specs=[
            pl.BlockSpec((_RB, _DF), rb),
            pl.BlockSpec((_RB, 16), rb),
        ],
        out_shape=[
            jax.ShapeDtypeStruct((_N, _DF), jnp.float32),
            jax.ShapeDtypeStruct((_N, 16), jnp.float32),
        ],
    )(x, W, S, Dm)


@functools.lru_cache(maxsize=None)
def _make_edge_pass(maps):
    mesh = plsc.VectorSubcoreMesh(core_axis_name="c", subcore_axis_name="s")

    @functools.partial(
        pl.kernel,
        mesh=mesh,
        compiler_params=pltpu.CompilerParams(use_tc_tiling_on_sc=False),
        out_type=jax.ShapeDtypeStruct((2, _NACC, _DF), jnp.float32),
        scratch_types=[
            pltpu.VMEM((6, _B), jnp.int32),
            pltpu.VMEM((6, _B), jnp.int32),
            pltpu.VMEM((3, _B, _DF), jnp.float32),
            pltpu.VMEM((3, _B, 16), jnp.float32),
            pltpu.VMEM_SHARED((_NACC, _DF), jnp.float32),
            pltpu.SemaphoreType.DMA,
            pltpu.SemaphoreType.DMA,
            pltpu.SemaphoreType.DMA,
            pltpu.SemaphoreType.DMA,
            pltpu.SemaphoreType.DMA,
            pltpu.SemaphoreType.DMA,
            pltpu.SemaphoreType.DMA,
            pltpu.SemaphoreType.DMA,
            pltpu.SemaphoreType.DMA,
            pltpu.SemaphoreType.DMA,
            pltpu.SemaphoreType.DMA,
            pltpu.SemaphoreType.DMA,
        ],
    )
    def k(t_hbm, adst_hbm, sidx_hbm, didx_hbm, acc_hbm,
          sidx_v, didx_v, rows_v, gd_v, acc_sh,
          sg0, sg1, sg2, ss0, ss1, ss2,
          si0, si1, si2, si3, si4, si5):
        cid = lax.axis_index("c")
        sid = lax.axis_index("s")
        wid = cid * 16 + sid
        sg = (sg0, sg1, sg2)
        ss = (ss0, ss1, ss2)
        si = (si0, si1, si2, si3, si4, si5)

        zero16 = jnp.zeros((16,), jnp.float32)

        def zrow(r, c):
            for kk in range(_DF // 16):
                rows_v[0, r, pl.ds(kk * 16, 16)] = zero16
            return c

        lax.fori_loop(0, _B, zrow, 0)
        rbase = sid * _RPT
        for j in range(7):
            pltpu.sync_copy(rows_v.at[0],
                            acc_sh.at[pl.ds(rbase + j * _B, _B)])
        pltpu.sync_copy(rows_v.at[0, pl.ds(0, _RPT - 7 * _B)],
                        acc_sh.at[pl.ds(rbase + 7 * _B, _RPT - 7 * _B)])
        plsc.subcore_barrier()

        def stage_idx(i, q):
            pltpu.async_copy(sidx_hbm.at[wid, i], sidx_v.at[q], si[q])
            pltpu.async_copy(didx_hbm.at[wid, i], didx_v.at[q], si[q])

        def wait_idx(q):
            pltpu.make_async_copy(sidx_hbm.at[wid, 0], sidx_v.at[q],
                                  si[q]).wait()
            pltpu.make_async_copy(didx_hbm.at[wid, 0], didx_v.at[q],
                                  si[q]).wait()

        def start_gathers(i, r, q):
            pltpu.async_copy(t_hbm.at[sidx_v.at[q]], rows_v.at[r], sg[r])
            pltpu.async_copy(adst_hbm.at[didx_v.at[q]], gd_v.at[r], sg[r])

        def wait_gathers(r):
            pltpu.make_async_copy(t_hbm.at[sidx_v.at[0]],
                                  rows_v.at[r], sg[r]).wait()
            pltpu.make_async_copy(adst_hbm.at[didx_v.at[0]],
                                  gd_v.at[r], sg[r]).wait()

        def drain_scatter(r):
            pltpu.make_async_copy(rows_v.at[r], acc_sh.at[didx_v.at[0]],
                                  ss[r]).wait()

        def compute(r, q):
            def edge(e, c2):
                a = rows_v[r, e, pl.ds(_D, 16)] + gd_v[r, e, :]
                a = jnp.where(a >= 0.0, a, a * 0.2)
                w = jnp.exp(a)
                rows_v[r, e, pl.ds(_D, 16)] = w
                ws = None
                for kk in range(8):
                    if kk == 0 or maps[kk] != maps[kk - 1]:
                        ws = jnp.broadcast_to(w[maps[kk]], (16,))
                    sl = pl.ds(kk * 16, 16)
                    rows_v[r, e, sl] = rows_v[r, e, sl] * ws
                return c2

            lax.fori_loop(0, _B, edge, 0, unroll=4)
            pltpu.async_copy(rows_v.at[r], acc_sh.at[didx_v.at[q]],
                             ss[r], add=True)

        def sec(i, qi, ri):
            r1, q1, q2 = (ri + 1) % 3, (qi + 1) % 6, (qi + 2) % 6
            drain_scatter(r1)
            wait_idx(q1)
            start_gathers(i + 1, r1, q1)
            stage_idx(i + 2, q2)
            wait_gathers(ri)
            compute(ri, qi)

        stage_idx(0, 0)
        wait_idx(0)
        start_gathers(0, 0, 0)
        stage_idx(1, 1)
        wait_idx(1)
        start_gathers(1, 1, 1)
        stage_idx(2, 2)
        wait_gathers(0)
        compute(0, 0)
        wait_idx(2)
        start_gathers(2, 2, 2)
        stage_idx(3, 3)
        wait_gathers(1)
        compute(1, 1)

        def body(j, c):
            i = j * 6 + 2
            for b in range(6):
                sec(i + b, (2 + b) % 6, (2 + b) % 3)
            return c

        lax.fori_loop(0, (_NB - 5) // 6, body, 0)
        i0 = _NB - 3
        r1, q1 = (i0 + 1) % 3, (i0 + 1) % 6
        drain_scatter(r1)
        wait_idx(q1)
        start_gathers(i0 + 1, r1, q1)
        stage_idx(i0 + 2, (i0 + 2) % 6)
        wait_gathers(i0 % 3)
        compute(i0 % 3, i0 % 6)
        i1 = _NB - 2
        r1, q1 = (i1 + 1) % 3, (i1 + 1) % 6
        drain_scatter(r1)
        wait_idx(q1)
        start_gathers(i1 + 1, r1, q1)
        wait_gathers(i1 % 3)
        compute(i1 % 3, i1 % 6)
        i2 = _NB - 1
        wait_gathers(i2 % 3)
        compute(i2 % 3, i2 % 6)
        for s in range(3):
            drain_scatter(s)
        plsc.subcore_barrier()

        for j in range(7):
            r0 = rbase + j * _B
            pltpu.sync_copy(acc_sh.at[pl.ds(r0, _B)],
                            acc_hbm.at[cid, pl.ds(r0, _B)])
        r0 = rbase + 7 * _B
        pltpu.sync_copy(acc_sh.at[pl.ds(r0, _RPT - 7 * _B)],
                        acc_hbm.at[cid, pl.ds(r0, _RPT - 7 * _B)])

    return k


def _mid_body(x_ref, t_ref, ad_ref, aa_ref, ab_ref,
              b0_ref, g_ref, be_ref, w1_ref, s1_ref, d1_ref, p_ref,
              t1_out, ad1_out):
    as16 = t_ref[:, _D:]
    a16 = as16 + ad_ref[:]
    wself = jnp.exp(jnp.where(a16 >= 0.0, a16, a16 * 0.2))
    den16 = aa_ref[0, :, _D:] + ab_ref[0, :, _D:] + wself
    wexp = jnp.dot(wself, p_ref[:], preferred_element_type=jnp.float32)
    dexp = jnp.dot(den16, p_ref[:], preferred_element_type=jnp.float32)
    num = aa_ref[0, :, :_D] + ab_ref[0, :, :_D] + wexp * t_ref[:, :_D]
    g = num / dexp + b0_ref[:]
    mu = jnp.mean(g, axis=1, keepdims=True)
    var = jnp.mean((g - mu) ** 2, axis=1, keepdims=True)
    ln = (g - mu) / jnp.sqrt(var + 1e-5) * g_ref[:] + be_ref[:]
    el = jnp.where(ln > 0.0, ln, jnp.exp(ln) - 1.0)
    h = el + x_ref[:]
    xl1 = jnp.dot(h, w1_ref[:], preferred_element_type=jnp.float32)
    t1_out[:, :_D] = xl1
    t1_out[:, _D:] = jnp.dot(xl1, s1_ref[:], preferred_element_type=jnp.float32)
    ad1_out[:] = jnp.dot(xl1, d1_ref[:], preferred_element_type=jnp.float32)


def _mid(x, t0, ad0, aa, ab, b0, g, be, W1, S1, D1, P):
    grid = (_N // _RB,)
    rb = lambda i: (i, 0)
    z = lambda i: (0, 0)
    return pl.pallas_call(
        _mid_body,
        grid=grid,
        in_specs=[
            pl.BlockSpec((_RB, _D), rb), pl.BlockSpec((_RB, _DF), rb),
            pl.BlockSpec((_RB, 16), rb),
            pl.BlockSpec((1, _RB, _DF), lambda i: (0, i, 0)),
            pl.BlockSpec((1, _RB, _DF), lambda i: (1, i, 0)),
            pl.BlockSpec((1, _D), z), pl.BlockSpec((1, _D), z),
            pl.BlockSpec((1, _D), z),
            pl.BlockSpec((_D, _D), z),
            pl.BlockSpec((_D, 16), z), pl.BlockSpec((_D, 16), z),
            pl.BlockSpec((16, _D), z),
        ],
        out_specs=[
            pl.BlockSpec((_RB, _DF), rb),
            pl.BlockSpec((_RB, 16), rb),
        ],
        out_shape=[
            jax.ShapeDtypeStruct((_N, _DF), jnp.float32),
            jax.ShapeDtypeStruct((_N, 16), jnp.float32),
        ],
    )(x, t0, ad0, aa, ab, b0, g, be, W1, S1, D1, P)


def _fin_body(t1_ref, ad_ref, aa_ref, ab_ref, b1_ref, pf_ref, out_ref):
    a16 = t1_ref[:, _D:] + ad_ref[:]
    wself = jnp.exp(jnp.where(a16 >= 0.0, a16, a16 * 0.2))
    den16 = aa_ref[0, :, _D:] + ab_ref[0, :, _D:] + wself
    w128 = jnp.dot(wself, pf_ref[:], preferred_element_type=jnp.float32)
    d128 = jnp.dot(den16, pf_ref[:], preferred_element_type=jnp.float32)
    num = aa_ref[0, :, :_D] + ab_ref[0, :, :_D] + w128 * t1_ref[:, :_D]
    out_ref[:] = num / d128 + b1_ref[:]


def _fin(t1, ad1, aa, ab, b1, PF):
    grid = (_N // _RB,)
    rb = lambda i: (i, 0)
    z = lambda i: (0, 0)
    return pl.pallas_call(
        _fin_body,
        grid=grid,
        in_specs=[
            pl.BlockSpec((_RB, _DF), rb), pl.BlockSpec((_RB, 16), rb),
            pl.BlockSpec((1, _RB, _DF), lambda i: (0, i, 0)),
            pl.BlockSpec((1, _RB, _DF), lambda i: (1, i, 0)),
            pl.BlockSpec((1, _D), z), pl.BlockSpec((16, _D), z),
        ],
        out_specs=pl.BlockSpec((_RB, _D), rb),
        out_shape=jax.ShapeDtypeStruct((_N, _D), jnp.float32),
    )(t1, ad1, aa, ab, b1, PF)


def kernel(x, edge_index, W0, att_src0, att_dst0, bias0, ln_g, ln_b,
           W1, att_src1, att_dst1, bias1):
    ei = edge_index.astype(jnp.int32)
    sidx = ei[0].reshape(_NW, _NB, _B)
    didx = ei[1].reshape(_NW, _NB, _B)

    eye8 = jnp.eye(8, 16, dtype=jnp.float32)
    S0 = (att_src0.reshape(8, 16)[:, :, None] * eye8[:, None, :]).reshape(128, 16)
    D0 = (att_dst0.reshape(8, 16)[:, :, None] * eye8[:, None, :]).reshape(128, 16)
    S1 = jnp.pad(att_src1.reshape(128, 1), ((0, 0), (0, 15)))
    D1 = jnp.pad(att_dst1.reshape(128, 1), ((0, 0), (0, 15)))
    P = jnp.repeat(jnp.eye(16, dtype=jnp.float32)[:, :8], 16, axis=1)
    PF = jnp.zeros((16, _D), jnp.float32).at[0].set(1.0)

    b0 = bias0.reshape(1, _D)
    b1 = bias1.reshape(1, _D)
    g = ln_g.reshape(1, _D)
    be = ln_b.reshape(1, _D)

    t0, ad0 = _prep(x, W0, S0, D0)
    acc0 = _make_edge_pass((0, 1, 2, 3, 4, 5, 6, 7))(t0, ad0, sidx, didx)
    t1, ad1 = _mid(x, t0, ad0, acc0, acc0, b0, g, be, W1, S1, D1, P)
    acc1 = _make_edge_pass((0, 0, 0, 0, 0, 0, 0, 0))(t1, ad1, sidx, didx)
    return _fin(t1, ad1, acc1, acc1, b1, PF)


# speedup vs baseline: 124.8422x; 1.3657x over previous
import functools

import jax
import jax.numpy as jnp
from jax import lax
from jax.experimental import pallas as pl
from jax.experimental.pallas import tpu as pltpu
from jax.experimental.pallas import tpu_sc as plsc

_N = 10000
_D = 128
_E = 320000
_RB = 2000
_NW = 32
_EPW = _E // _NW
_B = 80
_NB = _EPW // _B
_DF = _D + 16
_NACC = 10112
_RPT = _NACC // 16


def _prep_body(x_ref, w_ref, s_ref, d_ref, t_out, ad_out):
    xl = jnp.dot(x_ref[:], w_ref[:], preferred_element_type=jnp.float32)
    t_out[:, :_D] = xl
    t_out[:, _D:] = jnp.dot(xl, s_ref[:], preferred_element_type=jnp.float32)
    ad_out[:] = jnp.dot(xl, d_ref[:], preferred_element_type=jnp.float32)


def _prep(x, W, S, Dm):
    grid = (_N // _RB,)
    rb = lambda i: (i, 0)
    z = lambda i: (0, 0)
    return pl.pallas_call(
        _prep_body,
        grid=grid,
        in_specs=[
            pl.BlockSpec((_RB, _D), rb),
            pl.BlockSpec((_D, _D), z),
            pl.BlockSpec((_D, 16), z),
            pl.BlockSpec((_D, 16), z),
        ],
        out_specs=[
            pl.BlockSpec((_RB, _DF), rb),
            pl.BlockSpec((_RB, 16), rb),
        ],
        out_shape=[
            jax.ShapeDtypeStruct((_N, _DF), jnp.float32),
            jax.ShapeDtypeStruct((_N, 16), jnp.float32),
        ],
    )(x, W, S, Dm)


@functools.lru_cache(maxsize=None)
def _make_edge_pass(maps):
    mesh = plsc.VectorSubcoreMesh(core_axis_name="c", subcore_axis_name="s")

    @functools.partial(
        pl.kernel,
        mesh=mesh,
        compiler_params=pltpu.CompilerParams(use_tc_tiling_on_sc=False),
        out_type=jax.ShapeDtypeStruct((2, _NACC, _DF), jnp.float32),
        scratch_types=[
            pltpu.VMEM((6, _B), jnp.int32),
            pltpu.VMEM((6, _B), jnp.int32),
            pltpu.VMEM((3, _B, _DF), jnp.float32),
            pltpu.VMEM((3, _B, 16), jnp.float32),
            pltpu.VMEM_SHARED((_NACC, _DF), jnp.float32),
            pltpu.SemaphoreType.DMA,
            pltpu.SemaphoreType.DMA,
            pltpu.SemaphoreType.DMA,
            pltpu.SemaphoreType.DMA,
            pltpu.SemaphoreType.DMA,
            pltpu.SemaphoreType.DMA,
            pltpu.SemaphoreType.DMA,
            pltpu.SemaphoreType.DMA,
            pltpu.SemaphoreType.DMA,
            pltpu.SemaphoreType.DMA,
            pltpu.SemaphoreType.DMA,
            pltpu.SemaphoreType.DMA,
        ],
    )
    def k(t_hbm, adst_hbm, sidx_hbm, didx_hbm, acc_hbm,
          sidx_v, didx_v, rows_v, gd_v, acc_sh,
          sg0, sg1, sg2, ss0, ss1, ss2,
          si0, si1, si2, si3, si4, si5):
        cid = lax.axis_index("c")
        sid = lax.axis_index("s")
        wid = cid * 16 + sid
        sg = (sg0, sg1, sg2)
        ss = (ss0, ss1, ss2)
        si = (si0, si1, si2, si3, si4, si5)

        zero16 = jnp.zeros((16,), jnp.float32)

        def zrow(r, c):
            for kk in range(_DF // 16):
                rows_v[0, r, pl.ds(kk * 16, 16)] = zero16
            return c

        lax.fori_loop(0, _B, zrow, 0)
        rbase = sid * _RPT
        for j in range(7):
            pltpu.sync_copy(rows_v.at[0],
                            acc_sh.at[pl.ds(rbase + j * _B, _B)])
        pltpu.sync_copy(rows_v.at[0, pl.ds(0, _RPT - 7 * _B)],
                        acc_sh.at[pl.ds(rbase + 7 * _B, _RPT - 7 * _B)])
        plsc.subcore_barrier()

        def stage_idx(i, q):
            pltpu.async_copy(sidx_hbm.at[wid, i], sidx_v.at[q], si[q])
            pltpu.async_copy(didx_hbm.at[wid, i], didx_v.at[q], si[q])

        def wait_idx(q):
            pltpu.make_async_copy(sidx_hbm.at[wid, 0], sidx_v.at[q],
                                  si[q]).wait()
            pltpu.make_async_copy(didx_hbm.at[wid, 0], didx_v.at[q],
                                  si[q]).wait()

        def start_gathers(i, r, q):
            pltpu.async_copy(t_hbm.at[sidx_v.at[q]], rows_v.at[r], sg[r])
            pltpu.async_copy(adst_hbm.at[didx_v.at[q]], gd_v.at[r], sg[r])

        def wait_gathers(r):
            pltpu.make_async_copy(t_hbm.at[sidx_v.at[0]],
                                  rows_v.at[r], sg[r]).wait()
            pltpu.make_async_copy(adst_hbm.at[didx_v.at[0]],
                                  gd_v.at[r], sg[r]).wait()

        def drain_scatter(r):
            pltpu.make_async_copy(rows_v.at[r], acc_sh.at[didx_v.at[0]],
                                  ss[r]).wait()

        def compute(r, q):
            @plsc.parallel_loop(0, _B, 1, unroll=4)
            def edge(e):
                a = rows_v[r, e, pl.ds(_D, 16)] + gd_v[r, e, :]
                a = jnp.where(a >= 0.0, a, a * 0.2)
                w = jnp.exp(a)
                rows_v[r, e, pl.ds(_D, 16)] = w
                ws = None
                for kk in range(8):
                    if kk == 0 or maps[kk] != maps[kk - 1]:
                        ws = jnp.broadcast_to(w[maps[kk]], (16,))
                    sl = pl.ds(kk * 16, 16)
                    rows_v[r, e, sl] = rows_v[r, e, sl] * ws
            pltpu.async_copy(rows_v.at[r], acc_sh.at[didx_v.at[q]],
                             ss[r], add=True)

        def sec(i, qi, ri):
            r1, q1, q2 = (ri + 1) % 3, (qi + 1) % 6, (qi + 2) % 6
            drain_scatter(r1)
            wait_idx(q1)
            start_gathers(i + 1, r1, q1)
            stage_idx(i + 2, q2)
            wait_gathers(ri)
            compute(ri, qi)

        stage_idx(0, 0)
        wait_idx(0)
        start_gathers(0, 0, 0)
        stage_idx(1, 1)
        wait_idx(1)
        start_gathers(1, 1, 1)
        stage_idx(2, 2)
        wait_gathers(0)
        compute(0, 0)
        wait_idx(2)
        start_gathers(2, 2, 2)
        stage_idx(3, 3)
        wait_gathers(1)
        compute(1, 1)

        def body(j, c):
            i = j * 6 + 2
            for b in range(6):
                sec(i + b, (2 + b) % 6, (2 + b) % 3)
            return c

        lax.fori_loop(0, (_NB - 5) // 6, body, 0)
        i0 = _NB - 3
        r1, q1 = (i0 + 1) % 3, (i0 + 1) % 6
        drain_scatter(r1)
        wait_idx(q1)
        start_gathers(i0 + 1, r1, q1)
        stage_idx(i0 + 2, (i0 + 2) % 6)
        wait_gathers(i0 % 3)
        compute(i0 % 3, i0 % 6)
        i1 = _NB - 2
        r1, q1 = (i1 + 1) % 3, (i1 + 1) % 6
        drain_scatter(r1)
        wait_idx(q1)
        start_gathers(i1 + 1, r1, q1)
        wait_gathers(i1 % 3)
        compute(i1 % 3, i1 % 6)
        i2 = _NB - 1
        wait_gathers(i2 % 3)
        compute(i2 % 3, i2 % 6)
        for s in range(3):
            drain_scatter(s)
        plsc.subcore_barrier()

        for j in range(7):
            r0 = rbase + j * _B
            pltpu.sync_copy(acc_sh.at[pl.ds(r0, _B)],
                            acc_hbm.at[cid, pl.ds(r0, _B)])
        r0 = rbase + 7 * _B
        pltpu.sync_copy(acc_sh.at[pl.ds(r0, _RPT - 7 * _B)],
                        acc_hbm.at[cid, pl.ds(r0, _RPT - 7 * _B)])

    return k


def _mid_body(x_ref, t_ref, ad_ref, aa_ref, ab_ref,
              b0_ref, g_ref, be_ref, w1_ref, s1_ref, d1_ref, p_ref,
              t1_out, ad1_out):
    as16 = t_ref[:, _D:]
    a16 = as16 + ad_ref[:]
    wself = jnp.exp(jnp.where(a16 >= 0.0, a16, a16 * 0.2))
    den16 = aa_ref[0, :, _D:] + ab_ref[0, :, _D:] + wself
    wexp = jnp.dot(wself, p_ref[:], preferred_element_type=jnp.float32)
    dexp = jnp.dot(den16, p_ref[:], preferred_element_type=jnp.float32)
    num = aa_ref[0, :, :_D] + ab_ref[0, :, :_D] + wexp * t_ref[:, :_D]
    g = num / dexp + b0_ref[:]
    mu = jnp.mean(g, axis=1, keepdims=True)
    var = jnp.mean((g - mu) ** 2, axis=1, keepdims=True)
    ln = (g - mu) / jnp.sqrt(var + 1e-5) * g_ref[:] + be_ref[:]
    el = jnp.where(ln > 0.0, ln, jnp.exp(ln) - 1.0)
    h = el + x_ref[:]
    xl1 = jnp.dot(h, w1_ref[:], preferred_element_type=jnp.float32)
    t1_out[:, :_D] = xl1
    t1_out[:, _D:] = jnp.dot(xl1, s1_ref[:], preferred_element_type=jnp.float32)
    ad1_out[:] = jnp.dot(xl1, d1_ref[:], preferred_element_type=jnp.float32)


def _mid(x, t0, ad0, aa, ab, b0, g, be, W1, S1, D1, P):
    grid = (_N // _RB,)
    rb = lambda i: (i, 0)
    z = lambda i: (0, 0)
    return pl.pallas_call(
        _mid_body,
        grid=grid,
        in_specs=[
            pl.BlockSpec((_RB, _D), rb), pl.BlockSpec((_RB, _DF), rb),
            pl.BlockSpec((_RB, 16), rb),
            pl.BlockSpec((1, _RB, _DF), lambda i: (0, i, 0)),
            pl.BlockSpec((1, _RB, _DF), lambda i: (1, i, 0)),
            pl.BlockSpec((1, _D), z), pl.BlockSpec((1, _D), z),
            pl.BlockSpec((1, _D), z),
            pl.BlockSpec((_D, _D), z),
            pl.BlockSpec((_D, 16), z), pl.BlockSpec((_D, 16), z),
            pl.BlockSpec((16, _D), z),
        ],
        out_specs=[
            pl.BlockSpec((_RB, _DF), rb),
            pl.BlockSpec((_RB, 16), rb),
        ],
        out_shape=[
            jax.ShapeDtypeStruct((_N, _DF), jnp.float32),
            jax.ShapeDtypeStruct((_N, 16), jnp.float32),
        ],
    )(x, t0, ad0, aa, ab, b0, g, be, W1, S1, D1, P)


def _fin_body(t1_ref, ad_ref, aa_ref, ab_ref, b1_ref, pf_ref, out_ref):
    a16 = t1_ref[:, _D:] + ad_ref[:]
    wself = jnp.exp(jnp.where(a16 >= 0.0, a16, a16 * 0.2))
    den16 = aa_ref[0, :, _D:] + ab_ref[0, :, _D:] + wself
    w128 = jnp.dot(wself, pf_ref[:], preferred_element_type=jnp.float32)
    d128 = jnp.dot(den16, pf_ref[:], preferred_element_type=jnp.float32)
    num = aa_ref[0, :, :_D] + ab_ref[0, :, :_D] + w128 * t1_ref[:, :_D]
    out_ref[:] = num / d128 + b1_ref[:]


def _fin(t1, ad1, aa, ab, b1, PF):
    grid = (_N // _RB,)
    rb = lambda i: (i, 0)
    z = lambda i: (0, 0)
    return pl.pallas_call(
        _fin_body,
        grid=grid,
        in_specs=[
            pl.BlockSpec((_RB, _DF), rb), pl.BlockSpec((_RB, 16), rb),
            pl.BlockSpec((1, _RB, _DF), lambda i: (0, i, 0)),
            pl.BlockSpec((1, _RB, _DF), lambda i: (1, i, 0)),
            pl.BlockSpec((1, _D), z), pl.BlockSpec((16, _D), z),
        ],
        out_specs=pl.BlockSpec((_RB, _D), rb),
        out_shape=jax.ShapeDtypeStruct((_N, _D), jnp.float32),
    )(t1, ad1, aa, ab, b1, PF)


def kernel(x, edge_index, W0, att_src0, att_dst0, bias0, ln_g, ln_b,
           W1, att_src1, att_dst1, bias1):
    ei = edge_index.astype(jnp.int32)
    sidx = ei[0].reshape(_NW, _NB, _B)
    didx = ei[1].reshape(_NW, _NB, _B)

    eye8 = jnp.eye(8, 16, dtype=jnp.float32)
    S0 = (att_src0.reshape(8, 16)[:, :, None] * eye8[:, None, :]).reshape(128, 16)
    D0 = (att_dst0.reshape(8, 16)[:, :, None] * eye8[:, None, :]).reshape(128, 16)
    S1 = jnp.pad(att_src1.reshape(128, 1), ((0, 0), (0, 15)))
    D1 = jnp.pad(att_dst1.reshape(128, 1), ((0, 0), (0, 15)))
    P = jnp.repeat(jnp.eye(16, dtype=jnp.float32)[:, :8], 16, axis=1)
    PF = jnp.zeros((16, _D), jnp.float32).at[0].set(1.0)

    b0 = bias0.reshape(1, _D)
    b1 = bias1.reshape(1, _D)
    g = ln_g.reshape(1, _D)
    be = ln_b.reshape(1, _D)

    t0, ad0 = _prep(x, W0, S0, D0)
    acc0 = _make_edge_pass((0, 1, 2, 3, 4, 5, 6, 7))(t0, ad0, sidx, didx)
    t1, ad1 = _mid(x, t0, ad0, acc0, acc0, b0, g, be, W1, S1, D1, P)
    acc1 = _make_edge_pass((0, 0, 0, 0, 0, 0, 0, 0))(t1, ad1, sidx, didx)
    return _fin(t1, ad1, acc1, acc1, b1, PF)
